# Initial kernel scaffold; baseline (speedup 1.0000x reference)
import jax
import jax.numpy as jnp
from jax.experimental import pallas as pl


def kernel(hs, ds, Lmax, W1, b1, W2, b2):
    raise NotImplementedError("write your pallas kernel here")


# speedup vs baseline: 75.0578x; 75.0578x over previous
import functools

import jax
import jax.numpy as jnp
from jax.experimental import pallas as pl
from jax.experimental.pallas import tpu as pltpu


def _batch_body(L, Tmax, hs_ref, ds_ref, mult_ref, w1_ref, b1_ref, w2_ref,
                b2_ref, out_ref):
    f32 = jnp.float32
    ds_col = ds_ref[0]
    ds_f = ds_col.astype(f32)
    mult = mult_ref[0, 0]
    d = jnp.maximum(jnp.floor(ds_f * mult), 1.0)
    step = jnp.where(ds_col > 0, d, 0.0)

    t_iota = jax.lax.broadcasted_iota(jnp.int32, (Tmax, Tmax), 0)
    u_iota = jax.lax.broadcasted_iota(jnp.int32, (Tmax, Tmax), 1)
    tril = (u_iota <= t_iota).astype(jnp.bfloat16)
    ends = jax.lax.dot_general(
        tril, step.astype(jnp.bfloat16),
        (((1,), (0,)), ((), ())), preferred_element_type=f32)
    starts = ends - step

    fr = jax.lax.broadcasted_iota(jnp.int32, (Tmax, L), 1).astype(f32)
    member = ((fr >= starts) & (fr < ends)).astype(jnp.bfloat16)

    hs_b = hs_ref[0]
    hs_hi = hs_b.astype(jnp.bfloat16)
    hs_lo = (hs_b - hs_hi.astype(f32)).astype(jnp.bfloat16)
    sums = jax.lax.dot_general(
        member, hs_hi, (((1,), (0,)), ((), ())), preferred_element_type=f32)
    sums += jax.lax.dot_general(
        member, hs_lo, (((1,), (0,)), ((), ())), preferred_element_type=f32)

    recip = jnp.where(ds_col > 0, 1.0 / jnp.maximum(step, 1.0), 0.0)
    avg = sums * recip

    h = jax.lax.dot_general(
        avg, w1_ref[...], (((1,), (1,)), ((), ())), preferred_element_type=f32)
    h = jnp.maximum(h + b1_ref[...][None, 0, :], 0.0)
    o = jax.lax.dot_general(
        h, w2_ref[...], (((1,), (1,)), ((), ())), preferred_element_type=f32)
    o = jnp.maximum(o + b2_ref[...][None, 0, :], 0.0)
    out_ref[0] = o


def _run(hs, ds, Lmax, W1, b1, W2, b2):
    B, L, D = hs.shape
    Tmax = ds.shape[1]
    H = W2.shape[0]
    mult = (jnp.float32(L) / jnp.asarray(Lmax, jnp.float32)).reshape(1, 1)
    ds_col = ds.reshape(B, Tmax, 1)
    b1r = b1.reshape(1, -1)
    b2r = b2.reshape(1, -1)

    body = functools.partial(_batch_body, L, Tmax)
    out = pl.pallas_call(
        body,
        grid=(B,),
        in_specs=[
            pl.BlockSpec((1, L, D), lambda b: (b, 0, 0)),
            pl.BlockSpec((1, Tmax, 1), lambda b: (b, 0, 0)),
            pl.BlockSpec((1, 1), lambda b: (0, 0)),
            pl.BlockSpec(W1.shape, lambda b: (0, 0)),
            pl.BlockSpec(b1r.shape, lambda b: (0, 0)),
            pl.BlockSpec(W2.shape, lambda b: (0, 0)),
            pl.BlockSpec(b2r.shape, lambda b: (0, 0)),
        ],
        out_specs=pl.BlockSpec((1, Tmax, H), lambda b: (b, 0, 0)),
        out_shape=jax.ShapeDtypeStruct((B, Tmax, H), jnp.float32),
        compiler_params=pltpu.CompilerParams(
            dimension_semantics=("arbitrary",),
        ),
    )(hs, ds_col, mult, W1, b1r, W2, b2r)
    return out


def kernel(hs, ds, Lmax, W1, b1, W2, b2):
    return _run(hs, ds, Lmax, W1, b1, W2, b2)
